# Initial kernel scaffold; baseline (speedup 1.0000x reference)
import jax
import jax.numpy as jnp
from jax.experimental import pallas as pl


def kernel(x, edge_index, W1, b1, W2, b2):
    raise NotImplementedError("write your pallas kernel here")


# speedup vs baseline: 16.6679x; 16.6679x over previous
import functools

import jax
import jax.numpy as jnp
from jax import lax
from jax.experimental import pallas as pl
from jax.experimental.pallas import tpu as pltpu
from jax.experimental.pallas import tpu_sc as plsc

NC, NS, LANES = 2, 16, 16
NW = NC * NS
CHUNK = 128
N_NODES = 10000
N_EDGES = 320000
G = -(-N_EDGES // (NW * CHUNK))
EPAD = NW * G * CHUNK
NACC = 10240
ROWS_PER_TILE = NACC // NS
ZCOPIES = ROWS_PER_TILE // CHUNK


def _fill_rows(rows_v, d, value):
    vec = jnp.full((LANES,), value, jnp.float32)

    def body(i, _):
        def inner(j, _):
            rows_v[i, pl.ds(j * LANES, LANES)] = vec
            return 0
        return lax.fori_loop(0, d // LANES, inner, 0)

    lax.fori_loop(0, CHUNK, body, 0)


def _make_edge_scatter(d):
    mesh = plsc.VectorSubcoreMesh(core_axis_name="c", subcore_axis_name="s",
                                  num_cores=NC, num_subcores=NS)

    @functools.partial(
        pl.kernel,
        out_type=jax.ShapeDtypeStruct((NC, NACC, d), jnp.float32),
        mesh=mesh,
        scratch_types=[
            pltpu.VMEM((G, CHUNK), jnp.int32),
            pltpu.VMEM((G, CHUNK), jnp.int32),
            pltpu.VMEM((CHUNK, d), jnp.float32),
            pltpu.VMEM_SHARED((NACC, d), jnp.float32),
            pltpu.SemaphoreType.DMA,
        ],
        compiler_params=pltpu.CompilerParams(use_tc_tiling_on_sc=(d % 128 == 0)),
    )
    def k(y_hbm, src_hbm, dst_hbm, out_hbm, src_v, dst_v, rows_v, acc, sem):
        c = lax.axis_index("c")
        s = lax.axis_index("s")
        wid = s * NC + c
        base = s * ROWS_PER_TILE

        _fill_rows(rows_v, d, 0.0)

        def zero(kk, _):
            pltpu.sync_copy(rows_v, acc.at[pl.ds(base + kk * CHUNK, CHUNK)])
            return 0
        lax.fori_loop(0, ZCOPIES, zero, 0)

        pltpu.sync_copy(src_hbm.at[wid], src_v)
        pltpu.sync_copy(dst_hbm.at[wid], dst_v)
        plsc.subcore_barrier()

        def step(g, _):
            pltpu.async_copy(y_hbm.at[src_v.at[g]], rows_v, sem).wait()
            pltpu.sync_copy(rows_v, acc.at[dst_v.at[g]], add=True)
            return 0
        lax.fori_loop(0, G, step, 0)

        plsc.subcore_barrier()

        def wout(kk, _):
            r = base + kk * CHUNK
            pltpu.sync_copy(acc.at[pl.ds(r, CHUNK)], out_hbm.at[c, pl.ds(r, CHUNK)])
            return 0
        lax.fori_loop(0, ZCOPIES, wout, 0)

    return k


DEG_W = 16
BM = 2000


def _mm1_body(x_ref, w_ref, degp_ref, y_ref, dis_ref):
    deg = degp_ref[0][:, :1] + degp_ref[1][:, :1] + 1.0
    dis = lax.rsqrt(deg)
    hw = jnp.dot(x_ref[...], w_ref[...], preferred_element_type=jnp.float32)
    y_ref[...] = hw * dis
    dis_ref[...] = dis


def _mid_body(zp_ref, y1_ref, dis_ref, b1_ref, w2_ref, y2_ref):
    m = (zp_ref[0] + zp_ref[1] + y1_ref[...]) * dis_ref[...] + b1_ref[...]
    h = jnp.maximum(m, 0.0)
    hw = jnp.dot(h, w2_ref[...], preferred_element_type=jnp.float32)
    y2_ref[...] = hw * dis_ref[...]


def _out_body(zp_ref, y2_ref, dis_ref, b2_ref, o_ref):
    o_ref[...] = (zp_ref[0] + zp_ref[1] + y2_ref[...]) * dis_ref[...] + b2_ref[...]


def kernel(x, edge_index, W1, b1, W2, b2):
    n, d_in = x.shape
    d_hid = W1.shape[1]
    n_cls = W2.shape[1]

    src = edge_index[0]
    dst = edge_index[1]
    pad = EPAD - N_EDGES
    junk = n + (jnp.arange(pad, dtype=jnp.int32) % (NACC - n))
    srcp = jnp.concatenate([src, jnp.zeros((pad,), jnp.int32)]).reshape(NW, G, CHUNK)
    dstp = jnp.concatenate([dst, junk]).reshape(NW, G, CHUNK)

    ones_tab = jnp.ones((n, DEG_W), jnp.float32)
    degp = _make_edge_scatter(DEG_W)(ones_tab, srcp, dstp)

    grid = n // BM
    y1, dis = pl.pallas_call(
        _mm1_body,
        grid=(grid,),
        in_specs=[
            pl.BlockSpec((BM, d_in), lambda i: (i, 0)),
            pl.BlockSpec((d_in, d_hid), lambda i: (0, 0)),
            pl.BlockSpec((NC, BM, DEG_W), lambda i: (0, i, 0)),
        ],
        out_specs=[
            pl.BlockSpec((BM, d_hid), lambda i: (i, 0)),
            pl.BlockSpec((BM, 1), lambda i: (i, 0)),
        ],
        out_shape=[
            jax.ShapeDtypeStruct((n, d_hid), jnp.float32),
            jax.ShapeDtypeStruct((n, 1), jnp.float32),
        ],
    )(x, W1, degp)

    z1 = _make_edge_scatter(d_hid)(y1, srcp, dstp)

    y2 = pl.pallas_call(
        _mid_body,
        grid=(grid,),
        in_specs=[
            pl.BlockSpec((NC, BM, d_hid), lambda i: (0, i, 0)),
            pl.BlockSpec((BM, d_hid), lambda i: (i, 0)),
            pl.BlockSpec((BM, 1), lambda i: (i, 0)),
            pl.BlockSpec((1, d_hid), lambda i: (0, 0)),
            pl.BlockSpec((d_hid, n_cls), lambda i: (0, 0)),
        ],
        out_specs=pl.BlockSpec((BM, n_cls), lambda i: (i, 0)),
        out_shape=jax.ShapeDtypeStruct((n, n_cls), jnp.float32),
    )(z1, y1, dis, b1.reshape(1, d_hid), W2)

    z2 = _make_edge_scatter(n_cls)(y2, srcp, dstp)

    out = pl.pallas_call(
        _out_body,
        grid=(grid,),
        in_specs=[
            pl.BlockSpec((NC, BM, n_cls), lambda i: (0, i, 0)),
            pl.BlockSpec((BM, n_cls), lambda i: (i, 0)),
            pl.BlockSpec((BM, 1), lambda i: (i, 0)),
            pl.BlockSpec((1, n_cls), lambda i: (0, 0)),
        ],
        out_specs=pl.BlockSpec((BM, n_cls), lambda i: (i, 0)),
        out_shape=jax.ShapeDtypeStruct((n, n_cls), jnp.float32),
    )(z2, y2, dis, b2.reshape(1, n_cls))

    return out
